# baseline (speedup 1.0000x reference)
import jax
import jax.numpy as jnp
from jax import lax
from jax.experimental import pallas as pl
from jax.experimental.pallas import tpu as pltpu
from jax.experimental.pallas import tpu_sc as plsc

B, L, EMB, HID, TAG = 16384, 50, 64, 128, 1000
LP = 56
S = 2
LANES = 16
KV = EMB // LANES

_info = plsc.get_sparse_core_info()
NC, NS = _info.num_cores, _info.num_subcores
NW = NC * NS
SPW = B // NW
CH = SPW // S
IDXW = SPW * LP
CI = S * LP


def _sc_body(idx_hbm, emb_hbm, out_hbm, idx_v, rows0, rows1, outbuf, sem0, sem1):
    wid = lax.axis_index("s") * NC + lax.axis_index("c")
    pltpu.sync_copy(idx_hbm.at[pl.ds(wid * IDXW, IDXW)], idx_v)

    def start(c, buf, sem):
        pltpu.async_copy(emb_hbm.at[idx_v.at[pl.ds(c * CI, CI)]], buf, sem)

    def wait(buf, sem):
        pltpu.make_async_copy(emb_hbm.at[idx_v.at[pl.ds(0, CI)]], buf, sem).wait()

    def accum(c, buf):
        for t in range(S):
            accs = [buf[t * LP, pl.ds(k * LANES, LANES)] for k in range(KV)]
            for j in range(1, L):
                for k in range(KV):
                    accs[k] = accs[k] + buf[t * LP + j, pl.ds(k * LANES, LANES)]
            row = (c * S + t) * EMB
            for k in range(KV):
                outbuf[pl.ds(row + k * LANES, LANES)] = accs[k]

    start(0, rows0, sem0)
    start(1, rows1, sem1)

    @pl.loop(0, CH, step=2)
    def _(c):
        wait(rows0, sem0)
        accum(c, rows0)

        @pl.when(c + 2 < CH)
        def _():
            start(c + 2, rows0, sem0)

        wait(rows1, sem1)
        accum(c + 1, rows1)

        @pl.when(c + 3 < CH)
        def _():
            start(c + 3, rows1, sem1)

    pltpu.sync_copy(outbuf, out_hbm.at[pl.ds(wid * SPW * EMB, SPW * EMB)])


_sc_gather_sum = pl.kernel(
    _sc_body,
    out_type=jax.ShapeDtypeStruct((B * EMB,), jnp.float32),
    mesh=plsc.VectorSubcoreMesh(core_axis_name="c", subcore_axis_name="s"),
    compiler_params=pltpu.CompilerParams(use_tc_tiling_on_sc=False),
    scratch_types=[
        pltpu.VMEM((IDXW,), jnp.int32),
        pltpu.VMEM((CI, EMB), jnp.float32),
        pltpu.VMEM((CI, EMB), jnp.float32),
        pltpu.VMEM((SPW * EMB,), jnp.float32),
        pltpu.SemaphoreType.DMA,
        pltpu.SemaphoreType.DMA,
    ],
)

BS = 1024


def _mlp_body(x_ref, w1_ref, b1_ref, w2_ref, b2_ref, o_ref):
    x = x_ref[...]
    h = jnp.dot(x, w1_ref[...], preferred_element_type=jnp.float32) + b1_ref[...]
    h = jnp.maximum(h, 0.0)
    logits = jnp.dot(h, w2_ref[...], preferred_element_type=jnp.float32) + b2_ref[...]
    m = jnp.max(logits, axis=1, keepdims=True)
    shifted = logits - m
    lse = jnp.log(jnp.sum(jnp.exp(shifted), axis=1, keepdims=True))
    o_ref[...] = shifted - lse


_mlp = pl.pallas_call(
    _mlp_body,
    grid=(B // BS,),
    in_specs=[
        pl.BlockSpec((BS, EMB), lambda i: (i, 0)),
        pl.BlockSpec((EMB, HID), lambda i: (0, 0)),
        pl.BlockSpec((1, HID), lambda i: (0, 0)),
        pl.BlockSpec((HID, TAG), lambda i: (0, 0)),
        pl.BlockSpec((1, TAG), lambda i: (0, 0)),
    ],
    out_specs=pl.BlockSpec((BS, TAG), lambda i: (i, 0)),
    out_shape=jax.ShapeDtypeStruct((B, TAG), jnp.float32),
)


def kernel(inputs, emb, W1, b1, W2, b2):
    idx = jnp.pad(inputs.astype(jnp.int32), ((0, 0), (0, LP - L))).reshape(-1)
    embeds = _sc_gather_sum(idx, emb).reshape(B, EMB)
    return _mlp(embeds, W1.T, b1[None, :], W2.T, b2[None, :])


# speedup vs baseline: 2.8729x; 2.8729x over previous
import jax
import jax.numpy as jnp
from jax import lax
from jax.experimental import pallas as pl
from jax.experimental.pallas import tpu as pltpu
from jax.experimental.pallas import tpu_sc as plsc

B, L, EMB, HID, TAG = 16384, 50, 64, 128, 1000
S = 4
NBUF = 4
LANES = 16
KV = EMB // LANES

_info = plsc.get_sparse_core_info()
NC, NS = _info.num_cores, _info.num_subcores
NW = NC * NS
SPW = B // NW
CH = SPW // S
IDXW = SPW * L
CI = S * L


def _sc_body(idx_hbm, emb_hbm, out_hbm, idx_v, rows, outbuf, sems):
    wid = lax.axis_index("s") * NC + lax.axis_index("c")
    pltpu.sync_copy(idx_hbm.at[pl.ds(wid * IDXW, IDXW)], idx_v)

    def start(c, b):
        pltpu.async_copy(emb_hbm.at[idx_v.at[pl.ds(c * CI, CI)]], rows[b], sems[b])

    def wait(b):
        pltpu.make_async_copy(emb_hbm.at[idx_v.at[pl.ds(0, CI)]], rows[b], sems[b]).wait()

    def accum(c, b):
        buf = rows[b]
        for t in range(S):
            accs = [buf[t * L, pl.ds(k * LANES, LANES)] for k in range(KV)]
            for j in range(1, L):
                for k in range(KV):
                    accs[k] = accs[k] + buf[t * L + j, pl.ds(k * LANES, LANES)]
            row = (c * S + t) * EMB
            for k in range(KV):
                outbuf[pl.ds(row + k * LANES, LANES)] = accs[k]

    for b in range(NBUF):
        start(b, b)

    @pl.loop(0, CH, step=NBUF)
    def _(c):
        for b in range(NBUF):
            wait(b)
            accum(c + b, b)

            @pl.when(c + b + NBUF < CH)
            def _():
                start(c + b + NBUF, b)

    pltpu.sync_copy(outbuf, out_hbm.at[pl.ds(wid * SPW * EMB, SPW * EMB)])


_sc_gather_sum = pl.kernel(
    _sc_body,
    out_type=jax.ShapeDtypeStruct((B * EMB,), jnp.float32),
    mesh=plsc.VectorSubcoreMesh(core_axis_name="c", subcore_axis_name="s"),
    compiler_params=pltpu.CompilerParams(use_tc_tiling_on_sc=False),
    scratch_types=[
        pltpu.VMEM((IDXW,), jnp.int32),
        [pltpu.VMEM((CI, EMB), jnp.float32) for _ in range(NBUF)],
        pltpu.VMEM((SPW * EMB,), jnp.float32),
        [pltpu.SemaphoreType.DMA for _ in range(NBUF)],
    ],
)

BS = 1024


def _mlp_body(x_ref, w1_ref, b1_ref, w2_ref, b2_ref, o_ref):
    x = x_ref[...]
    h = jnp.dot(x, w1_ref[...], preferred_element_type=jnp.float32) + b1_ref[...]
    h = jnp.maximum(h, 0.0)
    logits = jnp.dot(h, w2_ref[...], preferred_element_type=jnp.float32) + b2_ref[...]
    m = jnp.max(logits, axis=1, keepdims=True)
    shifted = logits - m
    lse = jnp.log(jnp.sum(jnp.exp(shifted), axis=1, keepdims=True))
    o_ref[...] = shifted - lse


_mlp = pl.pallas_call(
    _mlp_body,
    grid=(B // BS,),
    in_specs=[
        pl.BlockSpec((BS, EMB), lambda i: (i, 0)),
        pl.BlockSpec((EMB, HID), lambda i: (0, 0)),
        pl.BlockSpec((1, HID), lambda i: (0, 0)),
        pl.BlockSpec((HID, TAG), lambda i: (0, 0)),
        pl.BlockSpec((1, TAG), lambda i: (0, 0)),
    ],
    out_specs=pl.BlockSpec((BS, TAG), lambda i: (i, 0)),
    out_shape=jax.ShapeDtypeStruct((B, TAG), jnp.float32),
)


def kernel(inputs, emb, W1, b1, W2, b2):
    idx = inputs.astype(jnp.int32).reshape(-1)
    embeds = _sc_gather_sum(idx, emb).reshape(B, EMB)
    return _mlp(embeds, W1.T, b1[None, :], W2.T, b2[None, :])


# speedup vs baseline: 2.8755x; 1.0009x over previous
import jax
import jax.numpy as jnp
from jax import lax
from jax.experimental import pallas as pl
from jax.experimental.pallas import tpu as pltpu
from jax.experimental.pallas import tpu_sc as plsc

B, L, EMB, HID, TAG = 16384, 50, 64, 128, 1000
S = 4
NBUF = 4
LANES = 16
KV = EMB // LANES

_info = plsc.get_sparse_core_info()
NC, NS = _info.num_cores, _info.num_subcores
NW = NC * NS
SPW = B // NW
CH = SPW // S
IDXW = SPW * L
CI = S * L


def _sc_body(idx_hbm, emb_hbm, out_hbm, idx_v, rows, outbuf, sems):
    wid = lax.axis_index("s") * NC + lax.axis_index("c")
    pltpu.sync_copy(idx_hbm.at[pl.ds(wid * IDXW, IDXW)], idx_v)

    def start(c, b):
        pltpu.async_copy(emb_hbm.at[idx_v.at[pl.ds(c * CI, CI)]], rows[b], sems[b])

    def wait(b):
        pltpu.make_async_copy(emb_hbm.at[idx_v.at[pl.ds(0, CI)]], rows[b], sems[b]).wait()

    def accum(c, b):
        buf = rows[b]
        for t in range(S):
            accs = [buf[t * L, pl.ds(k * LANES, LANES)] for k in range(KV)]
            for j in range(1, L):
                for k in range(KV):
                    accs[k] = accs[k] + buf[t * L + j, pl.ds(k * LANES, LANES)]
            row = (c * S + t) * EMB
            for k in range(KV):
                outbuf[pl.ds(row + k * LANES, LANES)] = accs[k]

    for b in range(NBUF):
        start(b, b)

    @pl.loop(0, CH, step=NBUF)
    def _(c):
        for b in range(NBUF):
            wait(b)
            accum(c + b, b)

            @pl.when(c + b + NBUF < CH)
            def _():
                start(c + b + NBUF, b)

    pltpu.sync_copy(outbuf, out_hbm.at[pl.ds(wid * SPW * EMB, SPW * EMB)])


_sc_gather_sum = pl.kernel(
    _sc_body,
    out_type=jax.ShapeDtypeStruct((B * EMB,), jnp.float32),
    mesh=plsc.VectorSubcoreMesh(core_axis_name="c", subcore_axis_name="s"),
    compiler_params=pltpu.CompilerParams(use_tc_tiling_on_sc=False),
    scratch_types=[
        pltpu.VMEM((IDXW,), jnp.int32),
        [pltpu.VMEM((CI, EMB), jnp.float32) for _ in range(NBUF)],
        pltpu.VMEM((SPW * EMB,), jnp.float32),
        [pltpu.SemaphoreType.DMA for _ in range(NBUF)],
    ],
)

BS = 1024


def _dot_nt(a, b):
    return lax.dot_general(a, b, (((1,), (1,)), ((), ())),
                           preferred_element_type=jnp.float32)


def _mlp_body(x_ref, w1_ref, b1_ref, w2_ref, b2_ref, o_ref):
    x = x_ref[...]
    h = _dot_nt(x, w1_ref[...]) + b1_ref[...]
    h = jnp.maximum(h, 0.0)
    logits = _dot_nt(h, w2_ref[...]) + b2_ref[...]
    m = jnp.max(logits, axis=1, keepdims=True)
    shifted = logits - m
    lse = jnp.log(jnp.sum(jnp.exp(shifted), axis=1, keepdims=True))
    o_ref[...] = shifted - lse


_mlp = pl.pallas_call(
    _mlp_body,
    grid=(B // BS,),
    in_specs=[
        pl.BlockSpec((BS, EMB), lambda i: (i, 0)),
        pl.BlockSpec((HID, EMB), lambda i: (0, 0)),
        pl.BlockSpec((1, HID), lambda i: (0, 0)),
        pl.BlockSpec((TAG, HID), lambda i: (0, 0)),
        pl.BlockSpec((1, TAG), lambda i: (0, 0)),
    ],
    out_specs=pl.BlockSpec((BS, TAG), lambda i: (i, 0)),
    out_shape=jax.ShapeDtypeStruct((B, TAG), jnp.float32),
)


def kernel(inputs, emb, W1, b1, W2, b2):
    idx = inputs.astype(jnp.int32).reshape(-1)
    embeds = _sc_gather_sum(idx, emb).reshape(B, EMB)
    return _mlp(embeds, W1, b1[None, :], W2, b2[None, :])


# speedup vs baseline: 3.0525x; 1.0616x over previous
import jax
import jax.numpy as jnp
from jax import lax
from jax.experimental import pallas as pl
from jax.experimental.pallas import tpu as pltpu
from jax.experimental.pallas import tpu_sc as plsc

B, L, EMB, HID, TAG = 16384, 50, 64, 128, 1000
VOCAB = 1000000
S = 4
NBUF = 4
LANES = 16
KV = EMB // LANES

_info = plsc.get_sparse_core_info()
NC, NS = _info.num_cores, _info.num_subcores
NW = NC * NS
SPW = B // NW
CH = SPW // S
IDXW = SPW * L
CI = S * L


def _sc_body(idx_hbm, emb_hbm, out_hbm, idx_v, rows, outbuf, sems):
    wid = lax.axis_index("s") * NC + lax.axis_index("c")
    pltpu.sync_copy(idx_hbm.at[pl.ds(wid * IDXW, IDXW)], idx_v)

    def start(c, b):
        pltpu.async_copy(emb_hbm.at[idx_v.at[pl.ds(c * CI, CI)]], rows[b], sems[b])

    def wait(b):
        pltpu.make_async_copy(emb_hbm.at[idx_v.at[pl.ds(0, CI)]], rows[b],
                              sems[b]).wait()

    def accum(c, b):
        buf = rows[b]
        for t in range(S):
            accs = [buf[t * L, pl.ds(k * LANES, LANES)] for k in range(KV)]
            for j in range(1, L):
                for k in range(KV):
                    accs[k] = accs[k] + buf[t * L + j, pl.ds(k * LANES, LANES)]
            row = (c * S + t) * EMB
            for k in range(KV):
                outbuf[pl.ds(row + k * LANES, LANES)] = accs[k]

    for b in range(NBUF):
        start(b, b)

    @pl.loop(0, CH, step=NBUF)
    def _(c):
        for b in range(NBUF):
            wait(b)
            accum(c + b, b)

            @pl.when(c + b + NBUF < CH)
            def _():
                start(c + b + NBUF, b)

    pltpu.sync_copy(outbuf, out_hbm.at[pl.ds(wid * SPW * EMB, SPW * EMB)])


_sc_gather_sum = pl.kernel(
    _sc_body,
    out_type=jax.ShapeDtypeStruct((B * EMB,), jnp.float32),
    mesh=plsc.VectorSubcoreMesh(core_axis_name="c", subcore_axis_name="s"),
    compiler_params=pltpu.CompilerParams(use_tc_tiling_on_sc=False),
    scratch_types=[
        pltpu.VMEM((IDXW,), jnp.int32),
        [pltpu.VMEM((CI, EMB), jnp.float32) for _ in range(NBUF)],
        pltpu.VMEM((SPW * EMB,), jnp.float32),
        [pltpu.SemaphoreType.DMA for _ in range(NBUF)],
    ],
)

BS = 1024


def _mlp_body(x_ref, w1_ref, b1_ref, w2_ref, b2_ref, o_ref):
    x = x_ref[...]
    ht = lax.dot_general(w1_ref[...], x, (((1,), (1,)), ((), ())),
                         preferred_element_type=jnp.float32) + b1_ref[...]
    ht = jnp.maximum(ht, 0.0)
    logits = lax.dot_general(w2_ref[...], ht, (((1,), (0,)), ((), ())),
                             preferred_element_type=jnp.float32) + b2_ref[...]
    m = jnp.max(logits, axis=0, keepdims=True)
    shifted = logits - m
    lse = jnp.log(jnp.sum(jnp.exp(shifted), axis=0, keepdims=True))
    o_ref[...] = shifted - lse


_mlp = pl.pallas_call(
    _mlp_body,
    grid=(B // BS,),
    in_specs=[
        pl.BlockSpec((BS, EMB), lambda i: (i, 0)),
        pl.BlockSpec((HID, EMB), lambda i: (0, 0)),
        pl.BlockSpec((HID, 1), lambda i: (0, 0)),
        pl.BlockSpec((TAG, HID), lambda i: (0, 0)),
        pl.BlockSpec((TAG, 1), lambda i: (0, 0)),
    ],
    out_specs=pl.BlockSpec((TAG, BS), lambda i: (0, i)),
    out_shape=jax.ShapeDtypeStruct((TAG, B), jnp.float32),
)


def kernel(inputs, emb, W1, b1, W2, b2):
    idx = inputs.astype(jnp.int32).reshape(-1)
    embeds = _sc_gather_sum(idx, emb).reshape(B, EMB)
    out_t = _mlp(embeds, W1, b1[:, None], W2, b2[:, None])
    return out_t.T
